# baseline (speedup 1.0000x reference)
import functools

import jax
import jax.numpy as jnp
from jax import lax
from jax.experimental import pallas as pl
from jax.experimental.pallas import tpu as pltpu
from jax.experimental.pallas import tpu_sc as plsc

_LANES = 16


@functools.lru_cache(maxsize=None)
def _gather_kernel(num_tokens, embed_dim, b_per_w, num_cores):
    mesh = plsc.VectorSubcoreMesh(core_axis_name="c", subcore_axis_name="s")
    pair_dim = 2 * embed_dim
    n_groups = b_per_w // _LANES

    @functools.partial(
        pl.kernel,
        mesh=mesh,
        out_type=jax.ShapeDtypeStruct((num_tokens, embed_dim), jnp.float32),
        scratch_types=[
            pltpu.VMEM((b_per_w,), jnp.int32),
            pltpu.VMEM((b_per_w,), jnp.int32),
            pltpu.VMEM((b_per_w // 2, pair_dim), jnp.float32),
            pltpu.VMEM((b_per_w, embed_dim), jnp.float32),
            pltpu.SemaphoreType.DMA,
        ],
        compiler_params=pltpu.CompilerParams(needs_layout_passes=False),
    )
    def body(idx_hbm, table_hbm, out_hbm, idx_v, row2_v, pairs_v, out_v, sem):
        wid = lax.axis_index("s") * num_cores + lax.axis_index("c")
        base = wid * b_per_w
        pltpu.sync_copy(idx_hbm.at[pl.ds(base, b_per_w)], idx_v)

        def shift_body(g, _):
            sl = pl.ds(g * _LANES, _LANES)
            row2_v[sl] = idx_v[sl] >> 1
            return 0

        lax.fori_loop(0, n_groups, shift_body, 0, unroll=4)

        half = b_per_w // 2
        half_groups = half // _LANES

        def process_chunk(c, _):
            cbase = c * half
            pltpu.async_copy(
                table_hbm.at[row2_v.at[pl.ds(cbase, half)]], pairs_v, sem
            ).wait()

            def sel_body(g, _):
                sl = pl.ds(cbase + g * _LANES, _LANES)
                tin_v = g * _LANES + lax.iota(jnp.int32, _LANES)
                tout_v = cbase + tin_v
                col0_v = (idx_v[sl] & 1) * embed_dim
                for e0 in range(0, embed_dim, 1):
                    vals = plsc.load_gather(pairs_v, [tin_v, col0_v + e0])
                    e_v = jnp.full((_LANES,), e0, jnp.int32)
                    plsc.store_scatter(out_v, [tout_v, e_v], vals)
                return 0

            lax.fori_loop(0, half_groups, sel_body, 0)
            return 0

        lax.fori_loop(0, 2, process_chunk, 0)
        pltpu.sync_copy(out_v, out_hbm.at[pl.ds(base, b_per_w)])

    return body


@jax.jit
def kernel(input_, weight):
    num_tokens = input_.shape[0]
    embed_dim = weight.shape[1]
    info = plsc.get_sparse_core_info()
    num_workers = info.num_cores * info.num_subcores
    b_per_w = num_tokens // num_workers
    idx = input_.astype(jnp.int32)
    table2 = weight.reshape(-1, 2 * embed_dim)
    fn = _gather_kernel(num_tokens, embed_dim, b_per_w, info.num_cores)
    return fn(idx, table2)


# speedup vs baseline: 1.7932x; 1.7932x over previous
import functools

import jax
import jax.numpy as jnp
from jax import lax
from jax.experimental import pallas as pl
from jax.experimental.pallas import tpu as pltpu
from jax.experimental.pallas import tpu_sc as plsc

_LAG = 64
_LANES = 16


@functools.lru_cache(maxsize=None)
def _gather_kernel(num_tokens, embed_dim, b_per_w, num_cores):
    mesh = plsc.VectorSubcoreMesh(core_axis_name="c", subcore_axis_name="s")

    @functools.partial(
        pl.kernel,
        mesh=mesh,
        out_type=jax.ShapeDtypeStruct((num_tokens, embed_dim), jnp.float32),
        scratch_types=[
            pltpu.VMEM((b_per_w,), jnp.int32),
            pltpu.VMEM((b_per_w, embed_dim), jnp.float32),
            pltpu.SemaphoreType.DMA,
        ],
        compiler_params=pltpu.CompilerParams(needs_layout_passes=False),
    )
    def body(idx_hbm, table_hbm, out_hbm, idx_v, out_v, sem):
        wid = lax.axis_index("s") * num_cores + lax.axis_index("c")
        base = wid * b_per_w
        pltpu.sync_copy(idx_hbm.at[pl.ds(base, b_per_w)], idx_v)

        n_groups = b_per_w // _LANES
        lag_groups = _LAG // _LANES

        def fire_group(g):
            vec = idx_v[pl.ds(g * _LANES, _LANES)]
            for j in range(_LANES):
                row = vec[j]
                pltpu.async_copy(
                    table_hbm.at[pl.ds(row, 1)],
                    out_v.at[pl.ds(g * _LANES + j, 1)],
                    sem,
                )

        def drain_group(g):
            for j in range(_LANES):
                pltpu.make_async_copy(
                    table_hbm.at[pl.ds(0, 1)],
                    out_v.at[pl.ds(g * _LANES + j, 1)],
                    sem,
                ).wait()

        def prime_body(g, _):
            fire_group(g)
            return 0

        lax.fori_loop(0, lag_groups, prime_body, 0)

        def steady_body(g, _):
            fire_group(g)
            drain_group(g - lag_groups)
            return 0

        lax.fori_loop(lag_groups, n_groups, steady_body, 0)

        def drain_body(g, _):
            drain_group(g)
            return 0

        lax.fori_loop(n_groups - lag_groups, n_groups, drain_body, 0)
        pltpu.sync_copy(out_v, out_hbm.at[pl.ds(base, b_per_w)])

    return body


@jax.jit
def kernel(input_, weight):
    num_tokens = input_.shape[0]
    embed_dim = weight.shape[1]
    info = plsc.get_sparse_core_info()
    num_workers = info.num_cores * info.num_subcores
    b_per_w = num_tokens // num_workers
    idx = input_.astype(jnp.int32)
    fn = _gather_kernel(num_tokens, embed_dim, b_per_w, info.num_cores)
    return fn(idx, weight)


# speedup vs baseline: 3.8350x; 2.1386x over previous
import functools

import jax
import jax.numpy as jnp
from jax import lax
from jax.experimental import pallas as pl
from jax.experimental.pallas import tpu as pltpu
from jax.experimental.pallas import tpu_sc as plsc

_LANES = 16


@functools.lru_cache(maxsize=None)
def _gather_kernel(num_tokens, embed_dim, vocab):
    CH = 512
    TAILW = 64
    NB = vocab // 128
    END_AL = NB * 128
    NW = 32
    BASE_BLK, EXTRA = divmod(NB, NW)
    MAXR = (BASE_BLK + 1) * 128
    NCH = -(-MAXR // CH)
    TPB = 16384
    NTG = num_tokens // _LANES

    mesh = plsc.VectorSubcoreMesh(core_axis_name="c", subcore_axis_name="s")

    @functools.partial(
        pl.kernel,
        mesh=mesh,
        out_type=jax.ShapeDtypeStruct((num_tokens * embed_dim,), jnp.float32),
        scratch_types=[
            pltpu.VMEM((TPB + _LANES,), jnp.int32),
            pltpu.VMEM((TPB + _LANES,), jnp.int32),
            pltpu.VMEM((embed_dim, 128), jnp.float32),
            pltpu.VMEM((_LANES * embed_dim,), jnp.float32),
            pltpu.VMEM((2, embed_dim, CH), jnp.float32),
            pltpu.SemaphoreType.DMA,
            pltpu.SemaphoreType.DMA,
        ],
        compiler_params=pltpu.CompilerParams(needs_layout_passes=False),
    )
    def body(idx_hbm, wt_hbm, tail_hbm, out_hbm, csub, plist, tailb, stage,
             buf, sem_in, sem_o):
        w = lax.axis_index("c") * _LANES + lax.axis_index("s")
        nblk = BASE_BLK + (w < EXTRA).astype(jnp.int32)
        vstart = 128 * (BASE_BLK * w + jnp.minimum(w, EXTRA))
        vend_al = vstart + 128 * nblk
        is_last = w == NW - 1
        vend_memb = jnp.where(is_last, vocab, vend_al)
        w_lim = vend_al - CH

        pltpu.sync_copy(idx_hbm.at[pl.ds(0, num_tokens)],
                        csub.at[pl.ds(0, num_tokens)])

        iota16 = lax.iota(jnp.int32, _LANES)
        dqs = [iota16 + 16 * q for q in range(embed_dim // _LANES)]

        def scan_body(g, off):
            vec = csub[pl.ds(g * _LANES, _LANES)]
            m = (vec >= vstart) & (vec < vend_memb)
            e = (vec - vstart) * 16384 + (g * _LANES + iota16)
            plsc.store_compressed(plist.at[pl.ds(off, _LANES)], e, mask=m)
            return off + plsc.all_reduce_population_count(m)[0]

        nmem = lax.fori_loop(0, NTG, scan_body, 0)
        nmg = (nmem + _LANES - 1) >> 4

        @pl.when(is_last)
        def _():
            pltpu.sync_copy(tail_hbm, tailb)

        def fire_chunk(kn):
            w0 = jnp.minimum(vstart + kn * CH, w_lim)
            par = kn & 1
            for p in range(8):
                pltpu.async_copy(
                    wt_hbm.at[pl.ds(p * 8, 8), pl.ds(w0, CH)],
                    buf.at[par, pl.ds(p * 8, 8), :],
                    sem_in,
                )

        def wait_chunk():
            for _p in range(8):
                pltpu.make_async_copy(
                    wt_hbm.at[pl.ds(0, 8), pl.ds(0, CH)],
                    buf.at[0, pl.ds(0, 8), :],
                    sem_in,
                ).wait()

        def emit_groups(ng, get_values):
            def group_body(gg, _):
                ee = csub[pl.ds(gg * _LANES, _LANES)]
                vl_vec = get_values[0](ee)
                t_vec = ee & 16383
                for j in range(_LANES):
                    vlj = vl_vec[j]
                    for q, dq in enumerate(dqs):
                        vals = get_values[1](vlj, q, dq)
                        stage[pl.ds(j * embed_dim + q * _LANES, _LANES)] = vals
                for j in range(_LANES):
                    pltpu.async_copy(
                        stage.at[pl.ds(j * embed_dim, embed_dim)],
                        out_hbm.at[pl.ds(t_vec[j] * embed_dim, embed_dim)],
                        sem_o,
                    )
                for j in range(_LANES):
                    pltpu.make_async_copy(
                        stage.at[pl.ds(0, embed_dim)],
                        out_hbm.at[pl.ds(0, embed_dim)],
                        sem_o,
                    ).wait()
                return 0

            lax.fori_loop(0, ng, group_body, 0)

        def rescan(lo_r, hi_r):
            def rbody(g, off):
                ee = plist[pl.ds(g * _LANES, _LANES)]
                vr = ee >> 14
                lane_ok = (g * _LANES + iota16) < nmem
                m = (vr >= lo_r) & (vr < hi_r) & lane_ok
                plsc.store_compressed(csub.at[pl.ds(off, _LANES)], ee, mask=m)
                return off + plsc.all_reduce_population_count(m)[0]

            n = lax.fori_loop(0, nmg, rbody, 0)

            @pl.when(n > 0)
            def _():
                e0 = csub[pl.ds(0, _LANES)][0]
                csub[pl.ds(n, _LANES)] = jnp.full((_LANES,), e0, jnp.int32)

            return jnp.where(n > 0, (n + _LANES - 1) >> 4, 0)

        def process_chunk(k):
            lo = vstart + k * CH
            hi = jnp.minimum(lo + CH, vend_al)
            w0 = jnp.minimum(lo, w_lim)
            par = k & 1
            par_v = jnp.full((_LANES,), par, jnp.int32)
            ng = rescan(lo - vstart, hi - vstart)
            shift = w0 - vstart

            def vl_of(ee):
                return (ee >> 14) - shift

            def val_of(vlj, q, dq):
                return plsc.load_gather(
                    buf, [par_v, dq, jnp.full((_LANES,), vlj, jnp.int32)])

            emit_groups(ng, (vl_of, val_of))

        fire_chunk(0)
        wait_chunk()

        def chunk_loop(k, _):
            @pl.when(k + 1 < NCH)
            def _():
                fire_chunk(k + 1)

            process_chunk(k)

            @pl.when(k + 1 < NCH)
            def _():
                wait_chunk()

            return 0

        lax.fori_loop(0, NCH, chunk_loop, 0)

        @pl.when(is_last)
        def _():
            ng = rescan(END_AL - vstart, vocab - vstart)
            tshift = END_AL - vstart

            def vl_of(ee):
                return (ee >> 14) - tshift

            def val_of(vlj, q, dq):
                return plsc.load_gather(
                    tailb, [dq, jnp.full((_LANES,), vlj, jnp.int32)])

            emit_groups(ng, (vl_of, val_of))

    return body


@jax.jit
def kernel(input_, weight):
    num_tokens = input_.shape[0]
    vocab, embed_dim = weight.shape
    idx = input_.astype(jnp.int32)
    end_al = (vocab // 128) * 128
    tail_tab = jnp.zeros((embed_dim, 128), jnp.float32)
    tail_tab = tail_tab.at[:, : vocab - end_al].set(weight[end_al:, :].T)
    fn = _gather_kernel(num_tokens, embed_dim, vocab)
    out_flat = fn(idx, weight.T, tail_tab)
    return out_flat.reshape(num_tokens, embed_dim)


# speedup vs baseline: 3.9247x; 1.0234x over previous
import functools

import jax
import jax.numpy as jnp
from jax import lax
from jax.experimental import pallas as pl
from jax.experimental.pallas import tpu as pltpu
from jax.experimental.pallas import tpu_sc as plsc

_LANES = 16


@functools.lru_cache(maxsize=None)
def _gather_kernel(num_tokens, embed_dim, vocab):
    CH = 512
    TAILW = 64
    NB = vocab // 128
    END_AL = NB * 128
    NW = 32
    BASE_BLK, EXTRA = divmod(NB, NW)
    MAXR = (BASE_BLK + 1) * 128
    NCH = -(-MAXR // CH)
    TPB = 16384
    NTG = num_tokens // _LANES

    mesh = plsc.VectorSubcoreMesh(core_axis_name="c", subcore_axis_name="s")

    @functools.partial(
        pl.kernel,
        mesh=mesh,
        out_type=jax.ShapeDtypeStruct((num_tokens, embed_dim), jnp.float32),
        scratch_types=[
            pltpu.VMEM((TPB + _LANES,), jnp.int32),
            pltpu.VMEM((TPB + _LANES,), jnp.int32),
            pltpu.VMEM((embed_dim, 128), jnp.float32),
            pltpu.VMEM((_LANES, embed_dim), jnp.float32),
            pltpu.VMEM((2, embed_dim, CH), jnp.float32),
            pltpu.SemaphoreType.DMA,
            pltpu.SemaphoreType.DMA,
        ],
        compiler_params=pltpu.CompilerParams(needs_layout_passes=False),
    )
    def body(idx_hbm, wt_hbm, tail_hbm, out_hbm, csub, plist, tailb, stage,
             buf, sem_in, sem_o):
        w = lax.axis_index("c") * _LANES + lax.axis_index("s")
        nblk = BASE_BLK + (w < EXTRA).astype(jnp.int32)
        vstart = 128 * (BASE_BLK * w + jnp.minimum(w, EXTRA))
        vend_al = vstart + 128 * nblk
        is_last = w == NW - 1
        vend_memb = jnp.where(is_last, vocab, vend_al)
        w_lim = vend_al - CH

        pltpu.sync_copy(idx_hbm.at[pl.ds(0, num_tokens)],
                        csub.at[pl.ds(0, num_tokens)])

        iota16 = lax.iota(jnp.int32, _LANES)
        dqs = [iota16 + 16 * q for q in range(embed_dim // _LANES)]

        def scan_body(g, off):
            vec = csub[pl.ds(g * _LANES, _LANES)]
            m = (vec >= vstart) & (vec < vend_memb)
            e = (vec - vstart) * 16384 + (g * _LANES + iota16)
            plsc.store_compressed(plist.at[pl.ds(off, _LANES)], e, mask=m)
            return off + plsc.all_reduce_population_count(m)[0]

        nmem = lax.fori_loop(0, NTG, scan_body, 0)
        nmg = (nmem + _LANES - 1) >> 4

        @pl.when(is_last)
        def _():
            pltpu.sync_copy(tail_hbm, tailb)

        def fire_chunk(kn):
            w0 = jnp.minimum(vstart + kn * CH, w_lim)
            par = kn & 1
            for p in range(8):
                pltpu.async_copy(
                    wt_hbm.at[pl.ds(p * 8, 8), pl.ds(w0, CH)],
                    buf.at[par, pl.ds(p * 8, 8), :],
                    sem_in,
                )

        def wait_chunk():
            for _p in range(8):
                pltpu.make_async_copy(
                    wt_hbm.at[pl.ds(0, 8), pl.ds(0, CH)],
                    buf.at[0, pl.ds(0, 8), :],
                    sem_in,
                ).wait()

        def emit_groups(ng, get_values):
            def group_body(gg, _):
                ee = csub[pl.ds(gg * _LANES, _LANES)]
                vl_vec = get_values[0](ee)
                t_vec = ee & 16383
                for j in range(_LANES):
                    vlj = vl_vec[j]
                    j_v = jnp.full((_LANES,), j, jnp.int32)
                    for q, dq in enumerate(dqs):
                        vals = get_values[1](vlj, q, dq)
                        plsc.store_scatter(stage, [j_v, 16 * q + iota16], vals)
                for j in range(_LANES):
                    pltpu.async_copy(
                        stage.at[pl.ds(j, 1), :],
                        out_hbm.at[pl.ds(t_vec[j], 1), :],
                        sem_o,
                    )
                for j in range(_LANES):
                    pltpu.make_async_copy(
                        stage.at[pl.ds(0, 1), :],
                        out_hbm.at[pl.ds(0, 1), :],
                        sem_o,
                    ).wait()
                return 0

            lax.fori_loop(0, ng, group_body, 0)

        def rescan(lo_r, hi_r):
            def rbody(g, off):
                ee = plist[pl.ds(g * _LANES, _LANES)]
                vr = ee >> 14
                lane_ok = (g * _LANES + iota16) < nmem
                m = (vr >= lo_r) & (vr < hi_r) & lane_ok
                plsc.store_compressed(csub.at[pl.ds(off, _LANES)], ee, mask=m)
                return off + plsc.all_reduce_population_count(m)[0]

            n = lax.fori_loop(0, nmg, rbody, 0)

            @pl.when(n > 0)
            def _():
                e0 = csub[pl.ds(0, _LANES)][0]
                csub[pl.ds(n, _LANES)] = jnp.full((_LANES,), e0, jnp.int32)

            return jnp.where(n > 0, (n + _LANES - 1) >> 4, 0)

        def process_chunk(k):
            lo = vstart + k * CH
            hi = jnp.minimum(lo + CH, vend_al)
            w0 = jnp.minimum(lo, w_lim)
            par = k & 1
            par_v = jnp.full((_LANES,), par, jnp.int32)
            ng = rescan(lo - vstart, hi - vstart)
            shift = w0 - vstart

            def vl_of(ee):
                return (ee >> 14) - shift

            def val_of(vlj, q, dq):
                return plsc.load_gather(
                    buf, [par_v, dq, jnp.full((_LANES,), vlj, jnp.int32)])

            emit_groups(ng, (vl_of, val_of))

        fire_chunk(0)
        wait_chunk()

        def chunk_loop(k, _):
            @pl.when(k + 1 < NCH)
            def _():
                fire_chunk(k + 1)

            process_chunk(k)

            @pl.when(k + 1 < NCH)
            def _():
                wait_chunk()

            return 0

        lax.fori_loop(0, NCH, chunk_loop, 0)

        @pl.when(is_last)
        def _():
            ng = rescan(END_AL - vstart, vocab - vstart)
            tshift = END_AL - vstart

            def vl_of(ee):
                return (ee >> 14) - tshift

            def val_of(vlj, q, dq):
                return plsc.load_gather(
                    tailb, [dq, jnp.full((_LANES,), vlj, jnp.int32)])

            emit_groups(ng, (vl_of, val_of))

    return body


@jax.jit
def kernel(input_, weight):
    num_tokens = input_.shape[0]
    vocab, embed_dim = weight.shape
    idx = input_.astype(jnp.int32)
    end_al = (vocab // 128) * 128
    tail_tab = jnp.zeros((embed_dim, 128), jnp.float32)
    tail_tab = tail_tab.at[:, : vocab - end_al].set(weight[end_al:, :].T)
    fn = _gather_kernel(num_tokens, embed_dim, vocab)
    return fn(idx, weight.T, tail_tab)
